# Initial kernel scaffold; baseline (speedup 1.0000x reference)
import jax
import jax.numpy as jnp
from jax import lax
from jax.experimental import pallas as pl
from jax.experimental.pallas import tpu as pltpu
from jax.experimental.pallas import tpu_sc as plsc

B, C, H, W = 64, 3, 512, 512
PH, PW = 64, 64

_NC, _NS = 2, 16
_NW = _NC * _NS
_UNITS = B * C
_PER_W = _UNITS // _NW


def _sc_scatter(img_ref, rows_ref, cols_ref, patch_ref,
                patch_v, rows_v, cols_v):
  pltpu.sync_copy(patch_ref, patch_v)
  pltpu.sync_copy(rows_ref, rows_v)
  pltpu.sync_copy(cols_ref, cols_v)

  wid = lax.axis_index("s") * _NC + lax.axis_index("c")

  for k in range(_PER_W):
    u = wid * _PER_W + k
    b = u // C
    ch = u % C
    r = rows_v[b]
    c = cols_v[b]
    pltpu.sync_copy(
        patch_v.at[ch],
        img_ref.at[b, ch, pl.ds(r, PH), pl.ds(c, PW)],
    )


def kernel(image, top_left_rows, top_left_cols, learned_patch):
  patch = learned_patch[0]

  scatter = pl.kernel(
      _sc_scatter,
      out_type=(),
      mesh=plsc.VectorSubcoreMesh(core_axis_name="c", subcore_axis_name="s"),
      scratch_types=[
          pltpu.VMEM((C, PH, PW), jnp.float32),
          pltpu.VMEM((B,), jnp.int32),
          pltpu.VMEM((B,), jnp.int32),
      ],
  )

  img_ref = jax.new_ref(image)
  scatter(img_ref, top_left_rows, top_left_cols, patch)
  return img_ref[...]


# speedup vs baseline: 4.9311x; 4.9311x over previous
import jax
import jax.numpy as jnp
from jax import lax
from jax.experimental import pallas as pl
from jax.experimental.pallas import tpu as pltpu

B, C, H, W = 64, 3, 512, 512
PH, PW = 64, 64


def _merge_body(rows_ref, cols_ref, img_ref, patch_ref, out_ref):
  b = pl.program_id(0)
  r = rows_ref[b]
  c = cols_ref[b]

  patch = patch_ref[0]
  canvas = jnp.pad(patch, ((0, 0), (0, H - PH), (0, W - PW)))
  canvas = pltpu.roll(canvas, c, axis=2)
  canvas = pltpu.roll(canvas, r, axis=1)

  row_ids = lax.broadcasted_iota(jnp.int32, (H, W), 0)
  col_ids = lax.broadcasted_iota(jnp.int32, (H, W), 1)
  inside = ((row_ids >= r) & (row_ids < r + PH)
            & (col_ids >= c) & (col_ids < c + PW))

  img = img_ref[0]
  out_ref[0] = jnp.where(inside[None], canvas, img)


def kernel(image, top_left_rows, top_left_cols, learned_patch):
  grid_spec = pltpu.PrefetchScalarGridSpec(
      num_scalar_prefetch=2,
      grid=(B,),
      in_specs=[
          pl.BlockSpec((1, C, H, W), lambda b, rows, cols: (b, 0, 0, 0)),
          pl.BlockSpec((1, C, PH, PW), lambda b, rows, cols: (0, 0, 0, 0)),
      ],
      out_specs=pl.BlockSpec((1, C, H, W), lambda b, rows, cols: (b, 0, 0, 0)),
  )
  return pl.pallas_call(
      _merge_body,
      grid_spec=grid_spec,
      out_shape=jax.ShapeDtypeStruct((B, C, H, W), jnp.float32),
  )(top_left_rows, top_left_cols, image, learned_patch)
